# baseline (device time: 7645 ns/iter reference)
import jax
import jax.numpy as jnp
from jax import lax
from jax.experimental import pallas as pl
from jax.experimental.pallas import tpu as pltpu


def kernel(x, pi):
    def body(x_ref, pi_ref, out_ref, xf32_ref, comm_ref,
             in_sem, out_sem, send_sem, recv_sem):
        my_x = lax.axis_index("x")
        my_y = lax.axis_index("y")
        my_z = lax.axis_index("z")
        swap = pi_ref[0] != 0

        @pl.when(jnp.logical_not(swap))
        def _():
            cp_in = pltpu.make_async_copy(x_ref, xf32_ref, in_sem)
            cp_in.start()
            cp_in.wait()
            comm_ref[...] = xf32_ref[...].astype(comm_ref.dtype)
            cp_out = pltpu.make_async_copy(comm_ref, out_ref, out_sem)
            cp_out.start()
            cp_out.wait()

        @pl.when(swap)
        def _():
            neighbor = (my_x, 1 - my_y, my_z)
            barrier_sem = pltpu.get_barrier_semaphore()
            pl.semaphore_signal(
                barrier_sem, inc=1,
                device_id=neighbor, device_id_type=pl.DeviceIdType.MESH,
            )
            cp_in = pltpu.make_async_copy(x_ref, xf32_ref, in_sem)
            cp_in.start()
            cp_in.wait()
            comm_ref[...] = xf32_ref[...].astype(comm_ref.dtype)
            pl.semaphore_wait(barrier_sem, 1)

            rdma = pltpu.make_async_remote_copy(
                src_ref=comm_ref,
                dst_ref=out_ref,
                send_sem=send_sem,
                recv_sem=recv_sem,
                device_id=neighbor,
                device_id_type=pl.DeviceIdType.MESH,
            )
            rdma.start()
            rdma.wait()

    return pl.pallas_call(
        body,
        out_shape=jax.ShapeDtypeStruct(x.shape, jnp.bfloat16),
        in_specs=[
            pl.BlockSpec(memory_space=pl.ANY),
            pl.BlockSpec(memory_space=pltpu.SMEM),
        ],
        out_specs=pl.BlockSpec(memory_space=pl.ANY),
        scratch_shapes=[
            pltpu.VMEM(x.shape, x.dtype),
            pltpu.VMEM(x.shape, jnp.bfloat16),
            pltpu.SemaphoreType.DMA,
            pltpu.SemaphoreType.DMA,
            pltpu.SemaphoreType.DMA,
            pltpu.SemaphoreType.DMA,
        ],
        compiler_params=pltpu.CompilerParams(collective_id=0),
    )(x, pi)
